# baseline (speedup 1.0000x reference)
import functools

import jax
import jax.numpy as jnp
from jax import lax
from jax.experimental import pallas as pl
from jax.experimental.pallas import tpu as pltpu
from jax.experimental.pallas import tpu_sc as plsc

E = 64
BS = 128
NUMEL = 32768
NW = 16
CHUNK = NUMEL // NW
SUB = CHUNK // 16
OUT_LEN = NUMEL + (E + 1) * (BS - 1)
NBLK = OUT_LEN // BS
FILL_W = 2560
LAST_FILL = OUT_LEN - 15 * FILL_W
FILL_BUF = 2624


def _body(flat_hbm, out_hbm, eid_hbm, ntp_hbm, exch_hbm,
          chunk, hist, basel, rankb, posb, valb, tot, alltot, cumoff, incl,
          fill, eidb, ntp, sem):
    w = lax.axis_index("s")
    lane = lax.iota(jnp.int32, 16)
    ones = jnp.ones((16,), jnp.int32)
    fifteen = jnp.full((16,), 15, jnp.int32)
    gidx0 = lane * SUB

    with jax.named_scope("load_chunk"):
        pltpu.sync_copy(flat_hbm.at[pl.ds(w * CHUNK, CHUNK)], chunk)
    with jax.named_scope("hist_zero"):
        for e in range(E):
            hist[e, :] = jnp.zeros((16,), jnp.int32)

    def p1(s, _):
        t = plsc.load_gather(chunk, [gidx0 + s])
        r = plsc.load_gather(hist, [t, lane])
        rankb[s // 8, pl.ds((s % 8) * 16, 16)] = r
        plsc.addupdate_scatter(hist, [t, lane], ones)
        return 0

    with jax.named_scope("p1_hist"):
        lax.fori_loop(0, SUB, p1, 0)

    with jax.named_scope("lane_cumsum"):
        for e in range(E):
            row = hist[e, :]
            basel[e, :] = plsc.cumsum(row) - row
        for g in range(4):
            eg = lane + g * 16
            tg = (plsc.load_gather(basel, [eg, fifteen])
                  + plsc.load_gather(hist, [eg, fifteen]))
            tot[pl.ds(g * 16, 16)] = tg
    with jax.named_scope("publish"):
        pltpu.sync_copy(tot, exch_hbm.at[w])

    fv = jnp.full((16,), NUMEL, jnp.int32)

    def pf(i, _):
        fill[pl.ds(i * 16, 16)] = fv
        return 0

    with jax.named_scope("fill"):
        lax.fori_loop(0, FILL_BUF // 16, pf, 0)

        @pl.when(w < 15)
        def _():
            pltpu.sync_copy(fill.at[pl.ds(0, FILL_W)],
                            out_hbm.at[pl.ds(w * FILL_W, FILL_W)])

        @pl.when(w == 15)
        def _():
            pltpu.sync_copy(fill.at[pl.ds(0, LAST_FILL)],
                            out_hbm.at[pl.ds(15 * FILL_W, LAST_FILL)])

    with jax.named_scope("barrier"):
        plsc.subcore_barrier()

    with jax.named_scope("reduce"):
        pltpu.sync_copy(exch_hbm, alltot)
    carry = jnp.int32(0)
    with jax.named_scope("global_cumsum"):
        for g in range(4):
            off = jnp.zeros((16,), jnp.int32)
            cnt = jnp.zeros((16,), jnp.int32)
            for wp in range(NW):
                row = alltot[wp, pl.ds(g * 16, 16)]
                cnt = cnt + row
                before = jnp.broadcast_to(wp < w, (16,))
                off = off + jnp.where(before, row, jnp.zeros((16,), jnp.int32))
            pad = ((cnt + (BS - 1)) // BS) * BS
            inc_ = plsc.cumsum(pad)
            excl = inc_ - pad + carry
            cumoff[pl.ds(g * 16, 16)] = excl + off
            incl[pl.ds(g * 16, 16)] = excl + pad
            carry = carry + jnp.sum(pad)
    total = carry

    base_val = w * CHUNK

    def p2(s, _):
        t = plsc.load_gather(chunk, [gidx0 + s])
        r = rankb[s // 8, pl.ds((s % 8) * 16, 16)]
        b = plsc.load_gather(basel, [t, lane])
        c = plsc.load_gather(cumoff, [t])
        p = jnp.clip(b + c + r, 0, OUT_LEN - 1)
        v = base_val + gidx0 + s
        posb[s // 8, pl.ds((s % 8) * 16, 16)] = p
        valb[s // 8, pl.ds((s % 8) * 16, 16)] = v
        return 0

    with jax.named_scope("p2_pos"):
        lax.fori_loop(0, SUB, p2, 0)

    with jax.named_scope("scatter"):
        descs = [pltpu.async_copy(valb.at[j], out_hbm.at[posb.at[j]], sem)
                 for j in range(16)]
        for d in descs:
            d.wait()

    with jax.named_scope("eid"):
        _eid_tail(w, lane, incl, total, eidb, eid_hbm, ntp, ntp_hbm)


def _eid_tail(w, lane, incl, total, eidb, eid_hbm, ntp, ntp_hbm):
    @pl.when(w < 10)
    def _():
        ivs = [incl[pl.ds(g * 16, 16)] for g in range(4)]
        for h in range(2):
            bs_vec = (w * 32 + h * 16 + lane) * BS
            acc = jnp.zeros((16,), jnp.int32)
            for e in range(E):
                ce = ivs[e // 16][e % 16]
                acc = acc + (bs_vec >= ce).astype(jnp.int32)
            acc = jnp.where(bs_vec < total, acc, jnp.zeros((16,), jnp.int32))
            eidb[pl.ds(h * 16, 16)] = acc
        pltpu.sync_copy(eidb, eid_hbm.at[pl.ds(w * 32, 32)])

    @pl.when(w == 0)
    def _():
        ntp[...] = jnp.broadcast_to(total, (16,))
        pltpu.sync_copy(ntp.at[pl.ds(0, 1)], ntp_hbm)


_sc_align = pl.kernel(
    _body,
    out_type=(jax.ShapeDtypeStruct((OUT_LEN,), jnp.int32),
              jax.ShapeDtypeStruct((NBLK,), jnp.int32),
              jax.ShapeDtypeStruct((1,), jnp.int32),
              jax.ShapeDtypeStruct((NW, E), jnp.int32)),
    mesh=plsc.VectorSubcoreMesh(core_axis_name="c", subcore_axis_name="s",
                                num_cores=1),
    compiler_params=pltpu.CompilerParams(needs_layout_passes=False),
    scratch_types=[
        pltpu.VMEM((CHUNK,), jnp.int32),
        pltpu.VMEM((E, 16), jnp.int32),
        pltpu.VMEM((E, 16), jnp.int32),
        pltpu.VMEM((16, SUB), jnp.int32),
        pltpu.VMEM((16, SUB), jnp.int32),
        pltpu.VMEM((16, SUB), jnp.int32),
        pltpu.VMEM((E,), jnp.int32),
        pltpu.VMEM((NW, E), jnp.int32),
        pltpu.VMEM((E,), jnp.int32),
        pltpu.VMEM((E,), jnp.int32),
        pltpu.VMEM((FILL_BUF,), jnp.int32),
        pltpu.VMEM((32,), jnp.int32),
        pltpu.VMEM((16,), jnp.int32),
        pltpu.SemaphoreType.DMA,
    ],
)


def kernel(topk_ids, num_experts, block_size):
    flat = topk_ids.reshape(-1).astype(jnp.int32)
    sorted_ids, expert_ids, ntp, _ = _sc_align(flat)
    return (sorted_ids, expert_ids, ntp)


# speedup vs baseline: 2.4261x; 2.4261x over previous
import jax
import jax.numpy as jnp
from jax import lax
from jax.experimental import pallas as pl
from jax.experimental.pallas import tpu as pltpu
from jax.experimental.pallas import tpu_sc as plsc

E = 64
BS = 128
NUMEL = 32768
NW = 16
CHUNK = NUMEL // NW
SUB = CHUNK // 16
OUT_LEN = NUMEL + (E + 1) * (BS - 1)
NBLK = OUT_LEN // BS
SH_LEN = 41024
FILL_W = 2560
LAST_FILL = OUT_LEN - 15 * FILL_W
FILL_BUF = 2624


def _body(flat_hbm, out_hbm, eid_hbm, ntp_hbm, exch_hbm,
          chunk, hist, basel, rankb, posb, valb, tot, alltot, cumoff, incl,
          fill, eidb, ntp, shout, sem):
    w = lax.axis_index("s")
    lane = lax.iota(jnp.int32, 16)
    ones = jnp.ones((16,), jnp.int32)
    fifteen = jnp.full((16,), 15, jnp.int32)
    gidx0 = lane * SUB

    pltpu.sync_copy(flat_hbm.at[pl.ds(w * CHUNK, CHUNK)], chunk)
    for e in range(E):
        hist[e, :] = jnp.zeros((16,), jnp.int32)

    def p1(s, _):
        t = plsc.load_gather(chunk, [gidx0 + s])
        r = plsc.load_gather(hist, [t, lane])
        rankb[s // 8, pl.ds((s % 8) * 16, 16)] = r
        plsc.addupdate_scatter(hist, [t, lane], ones)
        return 0

    lax.fori_loop(0, SUB, p1, 0)

    for e in range(E):
        row = hist[e, :]
        basel[e, :] = plsc.cumsum(row) - row
    for g in range(4):
        eg = lane + g * 16
        tg = (plsc.load_gather(basel, [eg, fifteen])
              + plsc.load_gather(hist, [eg, fifteen]))
        tot[pl.ds(g * 16, 16)] = tg
    pltpu.sync_copy(tot, exch_hbm.at[w])

    fv = jnp.full((16,), NUMEL, jnp.int32)

    def pf(i, _):
        fill[pl.ds(i * 16, 16)] = fv
        return 0

    lax.fori_loop(0, FILL_BUF // 16, pf, 0)
    pltpu.sync_copy(fill.at[pl.ds(0, FILL_W)],
                    shout.at[pl.ds(w * FILL_W, FILL_W)])

    @pl.when(w == 15)
    def _():
        pltpu.sync_copy(fill.at[pl.ds(0, SH_LEN - 16 * FILL_W)],
                        shout.at[pl.ds(16 * FILL_W, SH_LEN - 16 * FILL_W)])

    plsc.subcore_barrier()

    pltpu.sync_copy(exch_hbm, alltot)
    carry = jnp.int32(0)
    for g in range(4):
        off = jnp.zeros((16,), jnp.int32)
        cnt = jnp.zeros((16,), jnp.int32)
        for wp in range(NW):
            row = alltot[wp, pl.ds(g * 16, 16)]
            cnt = cnt + row
            before = jnp.broadcast_to(wp < w, (16,))
            off = off + jnp.where(before, row, jnp.zeros((16,), jnp.int32))
        pad = ((cnt + (BS - 1)) // BS) * BS
        inc_ = plsc.cumsum(pad)
        excl = inc_ - pad + carry
        cumoff[pl.ds(g * 16, 16)] = excl + off
        incl[pl.ds(g * 16, 16)] = excl + pad
        carry = carry + jnp.sum(pad)
    total = carry

    base_val = w * CHUNK

    def p2(s, _):
        t = plsc.load_gather(chunk, [gidx0 + s])
        r = rankb[s // 8, pl.ds((s % 8) * 16, 16)]
        b = plsc.load_gather(basel, [t, lane])
        c = plsc.load_gather(cumoff, [t])
        p = jnp.clip(b + c + r, 0, OUT_LEN - 1)
        v = base_val + gidx0 + s
        posb[s // 8, pl.ds((s % 8) * 16, 16)] = p
        valb[s // 8, pl.ds((s % 8) * 16, 16)] = v
        return 0

    lax.fori_loop(0, SUB, p2, 0)

    descs = [pltpu.async_copy(valb.at[j], shout.at[posb.at[j]], sem)
             for j in range(16)]
    for d in descs:
        d.wait()

    plsc.subcore_barrier()

    @pl.when(w < 15)
    def _():
        pltpu.sync_copy(shout.at[pl.ds(w * FILL_W, FILL_W)],
                        out_hbm.at[pl.ds(w * FILL_W, FILL_W)])

    @pl.when(w == 15)
    def _():
        pltpu.sync_copy(shout.at[pl.ds(15 * FILL_W, FILL_BUF)], fill)
        pltpu.sync_copy(fill.at[pl.ds(0, LAST_FILL)],
                        out_hbm.at[pl.ds(15 * FILL_W, LAST_FILL)])

    @pl.when(w < 10)
    def _():
        ivs = [incl[pl.ds(g * 16, 16)] for g in range(4)]
        for h in range(2):
            bs_vec = (w * 32 + h * 16 + lane) * BS
            acc = jnp.zeros((16,), jnp.int32)
            for e in range(E):
                ce = ivs[e // 16][e % 16]
                acc = acc + (bs_vec >= ce).astype(jnp.int32)
            acc = jnp.where(bs_vec < total, acc, jnp.zeros((16,), jnp.int32))
            eidb[pl.ds(h * 16, 16)] = acc
        pltpu.sync_copy(eidb, eid_hbm.at[pl.ds(w * 32, 32)])

    @pl.when(w == 0)
    def _():
        ntp[...] = jnp.broadcast_to(total, (16,))
        pltpu.sync_copy(ntp.at[pl.ds(0, 1)], ntp_hbm)


_sc_align = pl.kernel(
    _body,
    out_type=(jax.ShapeDtypeStruct((OUT_LEN,), jnp.int32),
              jax.ShapeDtypeStruct((NBLK,), jnp.int32),
              jax.ShapeDtypeStruct((1,), jnp.int32),
              jax.ShapeDtypeStruct((NW, E), jnp.int32)),
    mesh=plsc.VectorSubcoreMesh(core_axis_name="c", subcore_axis_name="s",
                                num_cores=1),
    compiler_params=pltpu.CompilerParams(needs_layout_passes=False),
    scratch_types=[
        pltpu.VMEM((CHUNK,), jnp.int32),
        pltpu.VMEM((E, 16), jnp.int32),
        pltpu.VMEM((E, 16), jnp.int32),
        pltpu.VMEM((16, SUB), jnp.int32),
        pltpu.VMEM((16, SUB), jnp.int32),
        pltpu.VMEM((16, SUB), jnp.int32),
        pltpu.VMEM((E,), jnp.int32),
        pltpu.VMEM((NW, E), jnp.int32),
        pltpu.VMEM((E,), jnp.int32),
        pltpu.VMEM((E,), jnp.int32),
        pltpu.VMEM((FILL_BUF,), jnp.int32),
        pltpu.VMEM((32,), jnp.int32),
        pltpu.VMEM((16,), jnp.int32),
        pltpu.VMEM_SHARED((SH_LEN,), jnp.int32),
        pltpu.SemaphoreType.DMA,
    ],
)


def kernel(topk_ids, num_experts, block_size):
    flat = topk_ids.reshape(-1).astype(jnp.int32)
    sorted_ids, expert_ids, ntp, _ = _sc_align(flat)
    return (sorted_ids, expert_ids, ntp)


# speedup vs baseline: 2.5846x; 1.0653x over previous
import jax
import jax.numpy as jnp
from jax import lax
from jax.experimental import pallas as pl
from jax.experimental.pallas import tpu as pltpu
from jax.experimental.pallas import tpu_sc as plsc

E = 64
BS = 128
NUMEL = 32768
NW = 16
CHUNK = NUMEL // NW
SUB = CHUNK // 16
OUT_LEN = NUMEL + (E + 1) * (BS - 1)
NBLK = OUT_LEN // BS
SH_LEN = 41024
FILL_W = 2560
LAST_FILL = OUT_LEN - 15 * FILL_W
FILL_BUF = 2624


def _body(flat_hbm, out_hbm, eid_hbm, ntp_hbm, exch_hbm,
          chunk, hist, basel, rankb, posb, valb, tot, alltot, cumoff, incl,
          fill, eidb, ntp, shout, sem, fsem):
    w = lax.axis_index("s")
    lane = lax.iota(jnp.int32, 16)
    ones = jnp.ones((16,), jnp.int32)
    fifteen = jnp.full((16,), 15, jnp.int32)
    gidx0 = lane * SUB

    pltpu.sync_copy(flat_hbm.at[pl.ds(w * CHUNK, CHUNK)], chunk)
    for e in range(E):
        hist[e, :] = jnp.zeros((16,), jnp.int32)

    fv = jnp.full((16,), NUMEL, jnp.int32)

    def pf(i, _):
        fill[pl.ds(i * 16, 16)] = fv
        return 0

    lax.fori_loop(0, FILL_BUF // 16, pf, 0)
    fill_desc = pltpu.async_copy(fill.at[pl.ds(0, FILL_W)],
                                 shout.at[pl.ds(w * FILL_W, FILL_W)], fsem)

    @pl.when(w == 15)
    def _():
        pltpu.sync_copy(fill.at[pl.ds(0, SH_LEN - 16 * FILL_W)],
                        shout.at[pl.ds(16 * FILL_W, SH_LEN - 16 * FILL_W)])

    def p1(s, _):
        t = plsc.load_gather(chunk, [gidx0 + s])
        r = plsc.load_gather(hist, [t, lane])
        rankb[s // 8, pl.ds((s % 8) * 16, 16)] = r
        plsc.addupdate_scatter(hist, [t, lane], ones)
        return 0

    lax.fori_loop(0, SUB, p1, 0)

    for e in range(E):
        row = hist[e, :]
        basel[e, :] = plsc.cumsum(row) - row
    for g in range(4):
        eg = lane + g * 16
        tg = (plsc.load_gather(basel, [eg, fifteen])
              + plsc.load_gather(hist, [eg, fifteen]))
        tot[pl.ds(g * 16, 16)] = tg
    pltpu.sync_copy(tot, exch_hbm.at[w])

    fill_desc.wait()
    plsc.subcore_barrier()

    pltpu.sync_copy(exch_hbm, alltot)
    carry = jnp.int32(0)
    for g in range(4):
        off = jnp.zeros((16,), jnp.int32)
        cnt = jnp.zeros((16,), jnp.int32)
        for wp in range(NW):
            row = alltot[wp, pl.ds(g * 16, 16)]
            cnt = cnt + row
            before = jnp.broadcast_to(wp < w, (16,))
            off = off + jnp.where(before, row, jnp.zeros((16,), jnp.int32))
        pad = ((cnt + (BS - 1)) // BS) * BS
        inc_ = plsc.cumsum(pad)
        excl = inc_ - pad + carry
        cumoff[pl.ds(g * 16, 16)] = excl + off
        incl[pl.ds(g * 16, 16)] = excl + pad
        carry = carry + jnp.sum(pad)
    total = carry

    base_val = w * CHUNK

    @plsc.parallel_loop(0, SUB, 1, unroll=4)
    def _(s):
        t = plsc.load_gather(chunk, [gidx0 + s])
        r = rankb[s // 8, pl.ds((s % 8) * 16, 16)]
        b = plsc.load_gather(basel, [t, lane])
        c = plsc.load_gather(cumoff, [t])
        p = jnp.clip(b + c + r, 0, OUT_LEN - 1)
        v = base_val + gidx0 + s
        posb[s // 8, pl.ds((s % 8) * 16, 16)] = p
        valb[s // 8, pl.ds((s % 8) * 16, 16)] = v

    descs = [pltpu.async_copy(valb.at[j], shout.at[posb.at[j]], sem)
             for j in range(16)]

    @pl.when(w < 10)
    def _():
        ivs = [incl[pl.ds(g * 16, 16)] for g in range(4)]
        for h in range(2):
            bs_vec = (w * 32 + h * 16 + lane) * BS
            acc = jnp.zeros((16,), jnp.int32)
            for e in range(E):
                ce = ivs[e // 16][e % 16]
                acc = acc + (bs_vec >= ce).astype(jnp.int32)
            acc = jnp.where(bs_vec < total, acc, jnp.zeros((16,), jnp.int32))
            eidb[pl.ds(h * 16, 16)] = acc
        pltpu.sync_copy(eidb, eid_hbm.at[pl.ds(w * 32, 32)])

    @pl.when(w == 0)
    def _():
        ntp[...] = jnp.broadcast_to(total, (16,))
        pltpu.sync_copy(ntp.at[pl.ds(0, 1)], ntp_hbm)

    for d in descs:
        d.wait()

    plsc.subcore_barrier()

    @pl.when(w < 15)
    def _():
        pltpu.sync_copy(shout.at[pl.ds(w * FILL_W, FILL_W)],
                        out_hbm.at[pl.ds(w * FILL_W, FILL_W)])

    @pl.when(w == 15)
    def _():
        pltpu.sync_copy(shout.at[pl.ds(15 * FILL_W, FILL_BUF)], fill)
        pltpu.sync_copy(fill.at[pl.ds(0, LAST_FILL)],
                        out_hbm.at[pl.ds(15 * FILL_W, LAST_FILL)])


_sc_align = pl.kernel(
    _body,
    out_type=(jax.ShapeDtypeStruct((OUT_LEN,), jnp.int32),
              jax.ShapeDtypeStruct((NBLK,), jnp.int32),
              jax.ShapeDtypeStruct((1,), jnp.int32),
              jax.ShapeDtypeStruct((NW, E), jnp.int32)),
    mesh=plsc.VectorSubcoreMesh(core_axis_name="c", subcore_axis_name="s",
                                num_cores=1),
    compiler_params=pltpu.CompilerParams(needs_layout_passes=False),
    scratch_types=[
        pltpu.VMEM((CHUNK,), jnp.int32),
        pltpu.VMEM((E, 16), jnp.int32),
        pltpu.VMEM((E, 16), jnp.int32),
        pltpu.VMEM((16, SUB), jnp.int32),
        pltpu.VMEM((16, SUB), jnp.int32),
        pltpu.VMEM((16, SUB), jnp.int32),
        pltpu.VMEM((E,), jnp.int32),
        pltpu.VMEM((NW, E), jnp.int32),
        pltpu.VMEM((E,), jnp.int32),
        pltpu.VMEM((E,), jnp.int32),
        pltpu.VMEM((FILL_BUF,), jnp.int32),
        pltpu.VMEM((32,), jnp.int32),
        pltpu.VMEM((16,), jnp.int32),
        pltpu.VMEM_SHARED((SH_LEN,), jnp.int32),
        pltpu.SemaphoreType.DMA,
        pltpu.SemaphoreType.DMA,
    ],
)


def kernel(topk_ids, num_experts, block_size):
    flat = topk_ids.reshape(-1).astype(jnp.int32)
    sorted_ids, expert_ids, ntp, _ = _sc_align(flat)
    return (sorted_ids, expert_ids, ntp)


# speedup vs baseline: 2.6549x; 1.0272x over previous
import jax
import jax.numpy as jnp
from jax import lax
from jax.experimental import pallas as pl
from jax.experimental.pallas import tpu as pltpu
from jax.experimental.pallas import tpu_sc as plsc

E = 64
BS = 128
NUMEL = 32768
NW = 16
CHUNK = NUMEL // NW
SUB = CHUNK // 16
OUT_LEN = NUMEL + (E + 1) * (BS - 1)
NBLK = OUT_LEN // BS
SH_LEN = 41024
FILL_W = 2560
LAST_FILL = OUT_LEN - 15 * FILL_W
FILL_BUF = 2624


def _body(flat_hbm, out_hbm, eid_hbm, ntp_hbm,
          chunk, hist, basel, rankb, posb, valb, tot, alltot, cumoff, incl,
          fill, eidb, ntp, shout, shex, sem, fsem):
    w = lax.axis_index("s")
    lane = lax.iota(jnp.int32, 16)
    ones = jnp.ones((16,), jnp.int32)
    fifteen = jnp.full((16,), 15, jnp.int32)
    gidx0 = lane * SUB

    pltpu.sync_copy(flat_hbm.at[pl.ds(w * CHUNK, CHUNK)], chunk)
    for e in range(E):
        hist[e, :] = jnp.zeros((16,), jnp.int32)

    fv = jnp.full((16,), NUMEL, jnp.int32)

    def pf(i, _):
        fill[pl.ds(i * 16, 16)] = fv
        return 0

    lax.fori_loop(0, FILL_BUF // 16, pf, 0)
    fill_desc = pltpu.async_copy(fill.at[pl.ds(0, FILL_W)],
                                 shout.at[pl.ds(w * FILL_W, FILL_W)], fsem)

    @pl.when(w == 15)
    def _():
        pltpu.sync_copy(fill.at[pl.ds(0, SH_LEN - 16 * FILL_W)],
                        shout.at[pl.ds(16 * FILL_W, SH_LEN - 16 * FILL_W)])

    def p1(s, _):
        t = plsc.load_gather(chunk, [gidx0 + s])
        r = plsc.load_gather(hist, [t, lane])
        rankb[s // 8, pl.ds((s % 8) * 16, 16)] = r
        plsc.addupdate_scatter(hist, [t, lane], ones)
        return 0

    lax.fori_loop(0, SUB, p1, 0)

    for e in range(E):
        row = hist[e, :]
        basel[e, :] = plsc.cumsum(row) - row
    for g in range(4):
        eg = lane + g * 16
        tg = (plsc.load_gather(basel, [eg, fifteen])
              + plsc.load_gather(hist, [eg, fifteen]))
        tot[pl.ds(g * 16, 16)] = tg
    pltpu.sync_copy(tot, shex.at[pl.ds(w * E, E)])

    fill_desc.wait()
    plsc.subcore_barrier()

    pltpu.sync_copy(shex, alltot)
    carry = jnp.int32(0)
    for g in range(4):
        off = jnp.zeros((16,), jnp.int32)
        cnt = jnp.zeros((16,), jnp.int32)
        for wp in range(NW):
            row = alltot[pl.ds(wp * E + g * 16, 16)]
            cnt = cnt + row
            before = jnp.broadcast_to(wp < w, (16,))
            off = off + jnp.where(before, row, jnp.zeros((16,), jnp.int32))
        pad = ((cnt + (BS - 1)) // BS) * BS
        inc_ = plsc.cumsum(pad)
        excl = inc_ - pad + carry
        cumoff[pl.ds(g * 16, 16)] = excl + off
        incl[pl.ds(g * 16, 16)] = excl + pad
        carry = carry + jnp.sum(pad)
    total = carry

    base_val = w * CHUNK

    @plsc.parallel_loop(0, SUB, 1, unroll=4)
    def _(s):
        t = plsc.load_gather(chunk, [gidx0 + s])
        r = rankb[s // 8, pl.ds((s % 8) * 16, 16)]
        b = plsc.load_gather(basel, [t, lane])
        c = plsc.load_gather(cumoff, [t])
        p = jnp.clip(b + c + r, 0, OUT_LEN - 1)
        v = base_val + gidx0 + s
        posb[s // 8, pl.ds((s % 8) * 16, 16)] = p
        valb[s // 8, pl.ds((s % 8) * 16, 16)] = v

    descs = [pltpu.async_copy(valb.at[j], shout.at[posb.at[j]], sem)
             for j in range(16)]

    @pl.when(w < 10)
    def _():
        ivs = [incl[pl.ds(g * 16, 16)] for g in range(4)]
        for h in range(2):
            bs_vec = (w * 32 + h * 16 + lane) * BS
            acc = jnp.zeros((16,), jnp.int32)
            for e in range(E):
                ce = ivs[e // 16][e % 16]
                acc = acc + (bs_vec >= ce).astype(jnp.int32)
            acc = jnp.where(bs_vec < total, acc, jnp.zeros((16,), jnp.int32))
            eidb[pl.ds(h * 16, 16)] = acc
        pltpu.sync_copy(eidb, eid_hbm.at[pl.ds(w * 32, 32)])

    @pl.when(w == 0)
    def _():
        ntp[...] = jnp.broadcast_to(total, (16,))
        pltpu.sync_copy(ntp.at[pl.ds(0, 1)], ntp_hbm)

    for d in descs:
        d.wait()

    plsc.subcore_barrier()

    @pl.when(w < 15)
    def _():
        pltpu.sync_copy(shout.at[pl.ds(w * FILL_W, FILL_W)],
                        out_hbm.at[pl.ds(w * FILL_W, FILL_W)])

    @pl.when(w == 15)
    def _():
        pltpu.sync_copy(shout.at[pl.ds(15 * FILL_W, FILL_BUF)], fill)
        pltpu.sync_copy(fill.at[pl.ds(0, LAST_FILL)],
                        out_hbm.at[pl.ds(15 * FILL_W, LAST_FILL)])


_sc_align = pl.kernel(
    _body,
    out_type=(jax.ShapeDtypeStruct((OUT_LEN,), jnp.int32),
              jax.ShapeDtypeStruct((NBLK,), jnp.int32),
              jax.ShapeDtypeStruct((1,), jnp.int32)),
    mesh=plsc.VectorSubcoreMesh(core_axis_name="c", subcore_axis_name="s",
                                num_cores=1),
    compiler_params=pltpu.CompilerParams(needs_layout_passes=False),
    scratch_types=[
        pltpu.VMEM((CHUNK,), jnp.int32),
        pltpu.VMEM((E, 16), jnp.int32),
        pltpu.VMEM((E, 16), jnp.int32),
        pltpu.VMEM((16, SUB), jnp.int32),
        pltpu.VMEM((16, SUB), jnp.int32),
        pltpu.VMEM((16, SUB), jnp.int32),
        pltpu.VMEM((E,), jnp.int32),
        pltpu.VMEM((NW * E,), jnp.int32),
        pltpu.VMEM((E,), jnp.int32),
        pltpu.VMEM((E,), jnp.int32),
        pltpu.VMEM((FILL_BUF,), jnp.int32),
        pltpu.VMEM((32,), jnp.int32),
        pltpu.VMEM((16,), jnp.int32),
        pltpu.VMEM_SHARED((SH_LEN,), jnp.int32),
        pltpu.VMEM_SHARED((NW * E,), jnp.int32),
        pltpu.SemaphoreType.DMA,
        pltpu.SemaphoreType.DMA,
    ],
)


def kernel(topk_ids, num_experts, block_size):
    flat = topk_ids.reshape(-1).astype(jnp.int32)
    return _sc_align(flat)
